# Initial kernel scaffold; baseline (speedup 1.0000x reference)
import jax
import jax.numpy as jnp
from jax.experimental import pallas as pl


def kernel(x, Wr, br, W1, b1, W2, b2):
    raise NotImplementedError("write your pallas kernel here")


# speedup vs baseline: 1.3263x; 1.3263x over previous
import functools

import jax
import jax.numpy as jnp
from jax.experimental import pallas as pl
from jax.experimental.pallas import tpu as pltpu

_T = 256


def _router_kernel(x_ref, wr_ref, br_ref, idx_ref, gate_ref):
    x = x_ref[...]
    wr = wr_ref[...]
    logits = jax.lax.dot_general(
        x, wr, (((1,), (1,)), ((), ())), preferred_element_type=jnp.float32)
    logits = logits + br_ref[...]
    m = jnp.max(logits, axis=1, keepdims=True)
    s = jnp.sum(jnp.exp(logits - m), axis=1, keepdims=True)
    idx_ref[...] = jnp.argmax(logits, axis=1)[:, None].astype(jnp.int32)
    gate_ref[...] = 1.0 / s


def _ffn_kernel(te_ref, xp_ref, w1_ref, b1_ref, w2_ref, b2_ref, gp_ref,
                out_ref, *, n_chunks):
    del te_ref
    xb = xp_ref[...]
    h = jax.lax.dot_general(
        xb, w1_ref[0], (((1,), (1,)), ((), ())),
        preferred_element_type=jnp.float32)
    h = jnp.maximum(h + b1_ref[0], 0.0)
    part = jax.lax.dot_general(
        h, w2_ref[0], (((1,), (1,)), ((), ())),
        preferred_element_type=jnp.float32)
    k = pl.program_id(1)

    @pl.when(k == 0)
    def _():
        out_ref[...] = part

    @pl.when(k > 0)
    def _():
        out_ref[...] += part

    @pl.when(k == n_chunks - 1)
    def _():
        out_ref[...] = (out_ref[...] + b2_ref[0]) * gp_ref[...]


@jax.jit
def kernel(x, Wr, br, W1, b1, W2, b2):
    N, D = x.shape
    E, F, _ = W1.shape
    T = _T
    G = N // T + E

    idx2, gate2 = pl.pallas_call(
        _router_kernel,
        out_shape=(
            jax.ShapeDtypeStruct((N, 1), jnp.int32),
            jax.ShapeDtypeStruct((N, 1), jnp.float32),
        ),
    )(x, Wr, br.reshape(1, E))
    idx = idx2[:, 0]
    gate = gate2[:, 0]

    counts = jnp.bincount(idx, length=E)
    tiles_e = (counts + T - 1) // T
    cum_tiles = jnp.cumsum(tiles_e)
    total_tiles = cum_tiles[-1]
    te_raw = jnp.searchsorted(cum_tiles, jnp.arange(G), side="right")
    te = jnp.minimum(te_raw, te_raw[total_tiles - 1]).astype(jnp.int32)
    tile_row_off = (cum_tiles - tiles_e) * T
    offsets = jnp.cumsum(counts) - counts

    order = jnp.argsort(idx)
    e_sorted = idx[order]
    slot_sorted = tile_row_off[e_sorted] + (jnp.arange(N) - offsets[e_sorted])
    src = jnp.full((G * T,), N, jnp.int32).at[slot_sorted].set(
        order.astype(jnp.int32))
    valid = src < N
    src_c = jnp.minimum(src, N - 1)
    xp = x[src_c]
    gp = jnp.where(valid, gate[src_c], 0.0)[:, None]

    K = 2
    Fc = F // K
    grid_spec = pltpu.PrefetchScalarGridSpec(
        num_scalar_prefetch=1,
        grid=(G, K),
        in_specs=[
            pl.BlockSpec((T, D), lambda i, k, te: (i, 0)),
            pl.BlockSpec((1, Fc, D), lambda i, k, te: (te[i], k, 0)),
            pl.BlockSpec((1, 1, Fc), lambda i, k, te: (te[i], 0, k)),
            pl.BlockSpec((1, D, Fc), lambda i, k, te: (te[i], 0, k)),
            pl.BlockSpec((1, 1, D), lambda i, k, te: (te[i], 0, 0)),
            pl.BlockSpec((T, 1), lambda i, k, te: (i, 0)),
        ],
        out_specs=pl.BlockSpec((T, D), lambda i, k, te: (i, 0)),
    )
    yp = pl.pallas_call(
        functools.partial(_ffn_kernel, n_chunks=K),
        grid_spec=grid_spec,
        out_shape=jax.ShapeDtypeStruct((G * T, D), jnp.float32),
    )(te, xp, W1, b1.reshape(E, 1, F), W2, b2.reshape(E, 1, D), gp)

    slot_of_token = jnp.zeros((N,), jnp.int32).at[order].set(
        slot_sorted.astype(jnp.int32))
    return yp[slot_of_token]
